# baseline (speedup 1.0000x reference)
import functools

import jax
import jax.numpy as jnp
from jax.experimental import pallas as pl

K_TOP = 32
N_KEYS = 100000
D = 128

QBLK = 1024
KBLK = 512
N_KEYS_PAD = ((N_KEYS + KBLK - 1) // KBLK) * KBLK


def _matmul_body(q_ref, k_ref, o_ref):
    ki = pl.program_id(1)
    qn = q_ref[...]
    kn = k_ref[...]
    s = jax.lax.dot_general(
        qn, kn, dimension_numbers=(((1,), (1,)), ((), ())),
        preferred_element_type=jnp.float32)
    col = ki * KBLK + jax.lax.broadcasted_iota(jnp.int32, s.shape, 1)
    o_ref[...] = jnp.where(col < N_KEYS, s, -1e30)


@functools.partial(jax.jit, static_argnames=())
def _similarities(query, key_memories):
    nq = query.shape[0]
    kpad = jnp.pad(key_memories, ((0, N_KEYS_PAD - N_KEYS), (0, 0)))
    grid = (nq // QBLK, N_KEYS_PAD // KBLK)
    return pl.pallas_call(
        _matmul_body,
        grid=grid,
        in_specs=[
            pl.BlockSpec((QBLK, D), lambda qi, ki: (qi, 0)),
            pl.BlockSpec((KBLK, D), lambda qi, ki: (ki, 0)),
        ],
        out_specs=pl.BlockSpec((QBLK, KBLK), lambda qi, ki: (qi, ki)),
        out_shape=jax.ShapeDtypeStruct((nq, N_KEYS_PAD), jnp.float32),
    )(query, kpad)


def kernel(query, key_memories, value_memories):
    qn = query / (jnp.linalg.norm(query, axis=-1, keepdims=True) + 1e-8)
    kn = key_memories / (
        jnp.linalg.norm(key_memories, axis=-1, keepdims=True) + 1e-8)
    sims = _similarities(qn, kn)
    scores, indices = jax.lax.top_k(sims, K_TOP)
    gathered = jnp.take(value_memories, indices, axis=0)
    return (scores, indices, gathered)


# speedup vs baseline: 11.2146x; 11.2146x over previous
import functools

import jax
import jax.numpy as jnp
from jax import lax
from jax.experimental import pallas as pl
from jax.experimental.pallas import tpu as pltpu
from jax.experimental.pallas import tpu_sc as plsc

K_TOP = 32
N_KEYS = 100000
D = 128
NQ = 4096

QBLK = 1024
KBLK = 512
SUB = KBLK // 128
N_KEYS_PAD = ((N_KEYS + KBLK - 1) // KBLK) * KBLK
NBLK = N_KEYS_PAD // 128

CAND_CAP = 48
SURV_CAP = 64
PAD_BLK = NBLK - 1
NEG = -1e30


def _mm_body(q_ref, k_ref, s3_ref, bm_ref):
    ki = pl.program_id(1)
    qn = q_ref[...]
    for b in range(SUB):
        kb = k_ref[b * 128:(b + 1) * 128, :]
        s = lax.dot_general(
            qn, kb, dimension_numbers=(((1,), (1,)), ((), ())),
            preferred_element_type=jnp.float32)
        col = ki * KBLK + b * 128 + lax.broadcasted_iota(jnp.int32, s.shape, 1)
        s = jnp.where(col < N_KEYS, s, NEG)
        s3_ref[b] = s
        bm_ref[0, b] = jnp.max(s, axis=1)


def _mm(qn, knp):
    grid = (NQ // QBLK, N_KEYS_PAD // KBLK)
    return pl.pallas_call(
        _mm_body,
        grid=grid,
        in_specs=[
            pl.BlockSpec((QBLK, D), lambda qi, ki: (qi, 0)),
            pl.BlockSpec((KBLK, D), lambda qi, ki: (ki, 0)),
        ],
        out_specs=[
            pl.BlockSpec((SUB, QBLK, D), lambda qi, ki: (ki, qi, 0)),
            pl.BlockSpec((1, SUB, QBLK), lambda qi, ki: (ki, 0, qi)),
        ],
        out_shape=[
            jax.ShapeDtypeStruct((NBLK, NQ, D), jnp.float32),
            jax.ShapeDtypeStruct((NBLK // SUB, SUB, NQ), jnp.float32),
        ],
    )(qn, knp)


TBLK = 256


def _thr_body(bm_ref, t_ref, scr_ref):
    scr_ref[...] = bm_ref[...]

    def body(i, carry):
        b = scr_ref[...]
        m = jnp.max(b, axis=0, keepdims=True)
        scr_ref[...] = jnp.where(b == m, -jnp.inf, b)
        return carry

    lax.fori_loop(0, K_TOP - 1, body, 0)
    t_ref[...] = jnp.max(scr_ref[...], axis=0, keepdims=True)


def _thresholds(bmT):
    return pl.pallas_call(
        _thr_body,
        grid=(NQ // TBLK,),
        in_specs=[pl.BlockSpec((NBLK, TBLK), lambda i: (0, i))],
        out_specs=pl.BlockSpec((1, TBLK), lambda i: (0, i)),
        out_shape=jax.ShapeDtypeStruct((1, NQ), jnp.float32),
        scratch_shapes=[pltpu.VMEM((NBLK, TBLK), jnp.float32)],
    )(bmT)


def _splat(x, dtype=jnp.int32):
    return jnp.full((16,), x, dtype)


def _scalar(vec):
    return lax.reduce_max(vec, axes=(0,))


def _sc_body(s3_hbm, bm_hbm, t_hbm, val_hbm, sc_o, ix_o, gv_o,
             tch, bmv, cand, gidx, gblk, sval, sidxv, srow, irow, vrows, sem):
    c = lax.axis_index("c")
    s = lax.axis_index("s")
    wid = s * 2 + c
    rows_per = NQ // 32
    base = wid * rows_per
    pltpu.sync_copy(t_hbm.at[pl.ds(base, rows_per)], tch)
    lane = lax.iota(jnp.int32, 16)

    def row_body(j, carry):
        r = base + j
        pltpu.sync_copy(bm_hbm.at[r], bmv)
        thr = plsc.load_gather(tch, [_splat(j)])

        for v in range(CAND_CAP // 16):
            cand[pl.ds(v * 16, 16)] = _splat(PAD_BLK)

        def cbody(v, off):
            vals = bmv[pl.ds(v * 16, 16)]
            m = vals >= thr
            ids = lane + v * 16
            plsc.store_compressed(cand.at[pl.ds(off, 16)], ids, mask=m)
            cnt = plsc.all_reduce_population_count(m)
            return off + _scalar(cnt)

        lax.fori_loop(0, NBLK // 16, cbody, jnp.int32(0))

        for v in range(CAND_CAP // 16):
            ids = cand[pl.ds(v * 16, 16)]
            gidx[pl.ds(v * 16, 16)] = ids * NQ + r
        pltpu.async_copy(s3_hbm.at[gidx], gblk, sem).wait()

        for v in range(SURV_CAP // 16):
            sval[pl.ds(v * 16, 16)] = jnp.full((16,), NEG, jnp.float32)
            sidxv[pl.ds(v * 16, 16)] = _splat(1 << 30)

        def fbody(v, soff):
            vb = v // 8
            l = v % 8
            vals = gblk[vb, pl.ds(l * 16, 16)]
            m = vals >= thr
            blk = plsc.load_gather(cand, [_splat(vb)])
            gi = blk * 128 + l * 16 + lane
            plsc.store_compressed(sval.at[pl.ds(soff, 16)], vals, mask=m)
            plsc.store_compressed(sidxv.at[pl.ds(soff, 16)], gi, mask=m)
            cnt = plsc.all_reduce_population_count(m)
            return soff + _scalar(cnt)

        lax.fori_loop(0, CAND_CAP * 8, fbody, jnp.int32(0))

        def rbody(i, carry):
            bval = plsc.load_gather(sval, [_splat(i)])
            bidx = plsc.load_gather(sidxv, [_splat(i)])
            rank = _splat(0)
            for v in range(SURV_CAP // 16):
                sv = sval[pl.ds(v * 16, 16)]
                si = sidxv[pl.ds(v * 16, 16)]
                beats = (sv > bval) | ((sv == bval) & (si < bidx))
                rank = rank + plsc.all_reduce_population_count(beats)
            emit = (lane == 0) & (rank < K_TOP)
            plsc.store_scatter(srow, [rank], bval, mask=emit)
            plsc.store_scatter(irow, [rank], bidx, mask=emit)
            return carry

        lax.fori_loop(0, SURV_CAP, rbody, 0)

        pltpu.async_copy(val_hbm.at[irow], vrows, sem).wait()
        pltpu.sync_copy(srow, sc_o.at[r])
        pltpu.sync_copy(irow, ix_o.at[r])
        pltpu.sync_copy(vrows, gv_o.at[r])
        return carry

    lax.fori_loop(0, rows_per, row_body, 0)


def _sc_select(s3v, bm, t, values):
    mesh = plsc.VectorSubcoreMesh(core_axis_name="c", subcore_axis_name="s")
    f = pl.kernel(
        _sc_body,
        mesh=mesh,
        compiler_params=pltpu.CompilerParams(needs_layout_passes=False),
        out_type=[
            jax.ShapeDtypeStruct((NQ, K_TOP), jnp.float32),
            jax.ShapeDtypeStruct((NQ, K_TOP), jnp.int32),
            jax.ShapeDtypeStruct((NQ, K_TOP, D), jnp.float32),
        ],
        scratch_types=[
            pltpu.VMEM((NQ // 32,), jnp.float32),
            pltpu.VMEM((NBLK,), jnp.float32),
            pltpu.VMEM((NBLK,), jnp.int32),
            pltpu.VMEM((CAND_CAP,), jnp.int32),
            pltpu.VMEM((CAND_CAP, D), jnp.float32),
            pltpu.VMEM((CAND_CAP * D,), jnp.float32),
            pltpu.VMEM((CAND_CAP * D,), jnp.int32),
            pltpu.VMEM((K_TOP,), jnp.float32),
            pltpu.VMEM((K_TOP,), jnp.int32),
            pltpu.VMEM((K_TOP, D), jnp.float32),
            pltpu.SemaphoreType.DMA,
        ],
    )
    return f(s3v, bm, t, values)


def kernel(query, key_memories, value_memories):
    qn = query / (jnp.linalg.norm(query, axis=-1, keepdims=True) + 1e-8)
    kn = key_memories / (
        jnp.linalg.norm(key_memories, axis=-1, keepdims=True) + 1e-8)
    knp = jnp.pad(kn, ((0, N_KEYS_PAD - N_KEYS), (0, 0)))
    s3, bmT3 = _mm(qn, knp)
    bmT = bmT3.reshape(NBLK, NQ)
    t = _thresholds(bmT).reshape(NQ)
    bm = bmT.T
    scores, indices, gathered = _sc_select(
        s3.reshape(NBLK * NQ, D), bm, t, value_memories)
    return (scores, indices, gathered)


# speedup vs baseline: 12.0613x; 1.0755x over previous
import functools

import jax
import jax.numpy as jnp
from jax import lax
from jax.experimental import pallas as pl
from jax.experimental.pallas import tpu as pltpu
from jax.experimental.pallas import tpu_sc as plsc

K_TOP = 32
N_KEYS = 100000
D = 128
NQ = 4096

QBLK = 1024
KBLK = 512
SUB = KBLK // 128
N_KEYS_PAD = ((N_KEYS + KBLK - 1) // KBLK) * KBLK
NBLK = N_KEYS_PAD // 128

CAND_CAP = 48
SURV_CAP = 48
PAD_BLK = NBLK - 1
NEG = -1e30


def _mm_body(q_ref, k_ref, s3_ref, bm_ref):
    ki = pl.program_id(1)
    qn = q_ref[...]
    for b in range(SUB):
        kb = k_ref[b * 128:(b + 1) * 128, :]
        s = lax.dot_general(
            qn, kb, dimension_numbers=(((1,), (1,)), ((), ())),
            preferred_element_type=jnp.float32)
        col = ki * KBLK + b * 128 + lax.broadcasted_iota(jnp.int32, s.shape, 1)
        s = jnp.where(col < N_KEYS, s, NEG)
        s3_ref[b] = s
        bm_ref[0, b] = jnp.max(s, axis=1)


def _mm(qn, knp):
    grid = (NQ // QBLK, N_KEYS_PAD // KBLK)
    return pl.pallas_call(
        _mm_body,
        grid=grid,
        in_specs=[
            pl.BlockSpec((QBLK, D), lambda qi, ki: (qi, 0)),
            pl.BlockSpec((KBLK, D), lambda qi, ki: (ki, 0)),
        ],
        out_specs=[
            pl.BlockSpec((SUB, QBLK, D), lambda qi, ki: (ki, qi, 0)),
            pl.BlockSpec((1, SUB, QBLK), lambda qi, ki: (ki, 0, qi)),
        ],
        out_shape=[
            jax.ShapeDtypeStruct((NBLK, NQ, D), jnp.float32),
            jax.ShapeDtypeStruct((NBLK // SUB, SUB, NQ), jnp.float32),
        ],
    )(qn, knp)


TBLK = 256


def _thr_body(bm_ref, t_ref, scr_ref):
    scr_ref[...] = bm_ref[...]

    def body(i, carry):
        b = scr_ref[...]
        m = jnp.max(b, axis=0, keepdims=True)
        scr_ref[...] = jnp.where(b == m, -jnp.inf, b)
        return carry

    lax.fori_loop(0, K_TOP - 1, body, 0)
    t_ref[...] = jnp.max(scr_ref[...], axis=0, keepdims=True)


def _thresholds(bmT):
    return pl.pallas_call(
        _thr_body,
        grid=(NQ // TBLK,),
        in_specs=[pl.BlockSpec((NBLK, TBLK), lambda i: (0, i))],
        out_specs=pl.BlockSpec((1, TBLK), lambda i: (0, i)),
        out_shape=jax.ShapeDtypeStruct((1, NQ), jnp.float32),
        scratch_shapes=[pltpu.VMEM((NBLK, TBLK), jnp.float32)],
    )(bmT)


def _splat(x, dtype=jnp.int32):
    return jnp.full((16,), x, dtype)


def _scalar(vec):
    return lax.reduce_max(vec, axes=(0,))


def _sc_body(s3_hbm, bm_hbm, t_hbm, val_hbm, sc_o, ix_o, gv_o,
             tch, bmv, cand, gidx, gblk, sval, sidxv, srow, irow, vrows, sem):
    c = lax.axis_index("c")
    s = lax.axis_index("s")
    wid = s * 2 + c
    rows_per = NQ // 32
    base = wid * rows_per
    pltpu.sync_copy(t_hbm.at[pl.ds(base, rows_per)], tch)
    lane = lax.iota(jnp.int32, 16)

    def row_body(j, carry):
        r = base + j
        pltpu.sync_copy(bm_hbm.at[r], bmv)
        thr = plsc.load_gather(tch, [_splat(j)])

        for v in range(CAND_CAP // 16):
            cand[pl.ds(v * 16, 16)] = _splat(PAD_BLK)

        def cbody(v, off):
            vals = bmv[pl.ds(v * 16, 16)]
            m = vals >= thr
            ids = lane + v * 16
            plsc.store_compressed(cand.at[pl.ds(off, 16)], ids, mask=m)
            cnt = plsc.all_reduce_population_count(m)
            return off + _scalar(cnt)

        cnt_c = lax.fori_loop(0, NBLK // 16, cbody, jnp.int32(0))
        n_blk = jnp.minimum(cnt_c, CAND_CAP)

        for v in range(CAND_CAP // 16):
            ids = cand[pl.ds(v * 16, 16)]
            gidx[pl.ds(v * 16, 16)] = ids * NQ + r
        pltpu.async_copy(s3_hbm.at[gidx], gblk, sem).wait()

        for v in range(SURV_CAP // 16):
            sval[pl.ds(v * 16, 16)] = jnp.full((16,), NEG, jnp.float32)
            sidxv[pl.ds(v * 16, 16)] = _splat(1 << 30)

        def fbody(vb, soff):
            blk = plsc.load_gather(cand, [_splat(vb)])
            gbase = blk * 128 + lane
            for l in range(8):
                vals = gblk[vb, pl.ds(l * 16, 16)]
                m = vals >= thr
                plsc.store_compressed(sval.at[pl.ds(soff, 16)], vals, mask=m)
                plsc.store_compressed(
                    sidxv.at[pl.ds(soff, 16)], gbase + l * 16, mask=m)
                cnt = plsc.all_reduce_population_count(m)
                soff = soff + _scalar(cnt)
            return soff

        n_surv = lax.fori_loop(0, n_blk, fbody, jnp.int32(0))

        def rbody(i, carry):
            bval = plsc.load_gather(sval, [_splat(i)])
            bidx = plsc.load_gather(sidxv, [_splat(i)])
            rank = _splat(0)
            for v in range(SURV_CAP // 16):
                sv = sval[pl.ds(v * 16, 16)]
                si = sidxv[pl.ds(v * 16, 16)]
                beats = (sv > bval) | ((sv == bval) & (si < bidx))
                rank = rank + plsc.all_reduce_population_count(beats)
            emit = (lane == 0) & (rank < K_TOP)
            plsc.store_scatter(srow, [rank], bval, mask=emit)
            plsc.store_scatter(irow, [rank], bidx, mask=emit)
            return carry

        lax.fori_loop(0, jnp.minimum(n_surv, SURV_CAP), rbody, 0)

        pltpu.async_copy(val_hbm.at[irow], vrows, sem).wait()
        pltpu.sync_copy(srow, sc_o.at[r])
        pltpu.sync_copy(irow, ix_o.at[r])
        pltpu.sync_copy(vrows, gv_o.at[r])
        return carry

    lax.fori_loop(0, rows_per, row_body, 0)


def _sc_select(s3v, bm, t, values):
    mesh = plsc.VectorSubcoreMesh(core_axis_name="c", subcore_axis_name="s")
    f = pl.kernel(
        _sc_body,
        mesh=mesh,
        compiler_params=pltpu.CompilerParams(needs_layout_passes=False),
        out_type=[
            jax.ShapeDtypeStruct((NQ, K_TOP), jnp.float32),
            jax.ShapeDtypeStruct((NQ, K_TOP), jnp.int32),
            jax.ShapeDtypeStruct((NQ, K_TOP, D), jnp.float32),
        ],
        scratch_types=[
            pltpu.VMEM((NQ // 32,), jnp.float32),
            pltpu.VMEM((NBLK,), jnp.float32),
            pltpu.VMEM((NBLK,), jnp.int32),
            pltpu.VMEM((CAND_CAP,), jnp.int32),
            pltpu.VMEM((CAND_CAP, D), jnp.float32),
            pltpu.VMEM((CAND_CAP * D,), jnp.float32),
            pltpu.VMEM((CAND_CAP * D,), jnp.int32),
            pltpu.VMEM((K_TOP,), jnp.float32),
            pltpu.VMEM((K_TOP,), jnp.int32),
            pltpu.VMEM((K_TOP, D), jnp.float32),
            pltpu.SemaphoreType.DMA,
        ],
    )
    return f(s3v, bm, t, values)


def kernel(query, key_memories, value_memories):
    qn = query / (jnp.linalg.norm(query, axis=-1, keepdims=True) + 1e-8)
    kn = key_memories / (
        jnp.linalg.norm(key_memories, axis=-1, keepdims=True) + 1e-8)
    knp = jnp.pad(kn, ((0, N_KEYS_PAD - N_KEYS), (0, 0)))
    s3, bmT3 = _mm(qn, knp)
    bmT = bmT3.reshape(NBLK, NQ)
    t = _thresholds(bmT).reshape(NQ)
    bm = bmT.T
    scores, indices, gathered = _sc_select(
        s3.reshape(NBLK * NQ, D), bm, t, value_memories)
    return (scores, indices, gathered)


# speedup vs baseline: 13.2112x; 1.0953x over previous
import functools

import jax
import jax.numpy as jnp
from jax import lax
from jax.experimental import pallas as pl
from jax.experimental.pallas import tpu as pltpu
from jax.experimental.pallas import tpu_sc as plsc

K_TOP = 32
N_KEYS = 100000
D = 128
NQ = 4096

QBLK = 1024
KBLK = 512
SUB = KBLK // 128
N_KEYS_PAD = ((N_KEYS + KBLK - 1) // KBLK) * KBLK
NBLK = N_KEYS_PAD // 128

CAND_CAP = 48
SURV_CAP = 48
PAD_BLK = NBLK - 1
NEG = -1e30


def _mm_body(q_ref, k_ref, s3_ref, bm_ref):
    ki = pl.program_id(1)
    qn = q_ref[...]
    for b in range(SUB):
        kb = k_ref[b * 128:(b + 1) * 128, :]
        s = lax.dot_general(
            qn, kb, dimension_numbers=(((1,), (1,)), ((), ())),
            preferred_element_type=jnp.float32)
        col = ki * KBLK + b * 128 + lax.broadcasted_iota(jnp.int32, s.shape, 1)
        s = jnp.where(col < N_KEYS, s, NEG)
        s3_ref[b] = s
        bm_ref[0, b] = jnp.max(s, axis=1)


def _mm(qn, knp):
    grid = (NQ // QBLK, N_KEYS_PAD // KBLK)
    return pl.pallas_call(
        _mm_body,
        grid=grid,
        in_specs=[
            pl.BlockSpec((QBLK, D), lambda qi, ki: (qi, 0)),
            pl.BlockSpec((KBLK, D), lambda qi, ki: (ki, 0)),
        ],
        out_specs=[
            pl.BlockSpec((SUB, QBLK, D), lambda qi, ki: (ki, qi, 0)),
            pl.BlockSpec((1, SUB, QBLK), lambda qi, ki: (ki, 0, qi)),
        ],
        out_shape=[
            jax.ShapeDtypeStruct((NBLK, NQ, D), jnp.float32),
            jax.ShapeDtypeStruct((NBLK // SUB, SUB, NQ), jnp.float32),
        ],
    )(qn, knp)


TBLK = 256


def _thr_body(bm_ref, t_ref, scr_ref):
    scr_ref[...] = bm_ref[...]

    def body(i, carry):
        b = scr_ref[...]
        m = jnp.max(b, axis=0, keepdims=True)
        scr_ref[...] = jnp.where(b == m, -jnp.inf, b)
        return carry

    lax.fori_loop(0, K_TOP - 1, body, 0)
    t_ref[...] = jnp.max(scr_ref[...], axis=0, keepdims=True)


def _thresholds(bmT):
    return pl.pallas_call(
        _thr_body,
        grid=(NQ // TBLK,),
        in_specs=[pl.BlockSpec((NBLK, TBLK), lambda i: (0, i))],
        out_specs=pl.BlockSpec((1, TBLK), lambda i: (0, i)),
        out_shape=jax.ShapeDtypeStruct((1, NQ), jnp.float32),
        scratch_shapes=[pltpu.VMEM((NBLK, TBLK), jnp.float32)],
    )(bmT)


def _splat(x, dtype=jnp.int32):
    return jnp.full((16,), x, dtype)


def _scalar(vec):
    return lax.reduce_max(vec, axes=(0,))


RB = 8
CSTRIDE = 64
SSTRIDE = 64


def _sc_body(s3_hbm, bm_hbm, t_hbm, val_hbm, sc_o, ix_o, gv_o,
             tch, bmv8, cand8, cnt8, gidx8, gblk8, sval8, sidx8,
             srow8, irow8, vrows8, sem):
    c = lax.axis_index("c")
    s = lax.axis_index("s")
    wid = s * 2 + c
    rows_per = NQ // 32
    base = wid * rows_per
    pltpu.sync_copy(t_hbm.at[pl.ds(base, rows_per)], tch)
    lane = lax.iota(jnp.int32, 16)

    def batch_body(bi, carry0):
        r0 = base + bi * RB
        pltpu.sync_copy(bm_hbm.at[pl.ds(r0, RB)], bmv8)

        def pass1(lr, carry):
            r = r0 + lr
            thr = plsc.load_gather(tch, [_splat(bi * RB + lr)])
            for v in range(CAND_CAP // 16):
                cand8[lr, pl.ds(v * 16, 16)] = _splat(PAD_BLK)

            def cbody(v, off):
                vals = bmv8[lr, pl.ds(v * 16, 16)]
                m = vals >= thr
                ids = lane + v * 16
                plsc.store_compressed(
                    cand8.at[lr, pl.ds(jnp.minimum(off, CAND_CAP), 16)],
                    ids, mask=m)
                cnt = plsc.all_reduce_population_count(m)
                return off + _scalar(cnt)

            cnt_c = lax.fori_loop(0, NBLK // 16, cbody, jnp.int32(0))
            plsc.store_scatter(cnt8, [_splat(lr)],
                               jnp.minimum(_splat(cnt_c), CAND_CAP),
                               mask=lane == 0)
            for v in range(CAND_CAP // 16):
                ids = cand8[lr, pl.ds(v * 16, 16)]
                gidx8[lr, pl.ds(v * 16, 16)] = ids * NQ + r
            return carry

        lax.fori_loop(0, RB, pass1, 0)

        cps = [pltpu.async_copy(s3_hbm.at[gidx8.at[lr]], gblk8.at[lr], sem)
               for lr in range(RB)]
        for cp in cps:
            cp.wait()

        def pass2(lr, carry):
            thr = plsc.load_gather(tch, [_splat(bi * RB + lr)])
            n_blk = _scalar(plsc.load_gather(cnt8, [_splat(lr)]))
            for v in range(SURV_CAP // 16):
                sval8[lr, pl.ds(v * 16, 16)] = jnp.full((16,), NEG,
                                                        jnp.float32)
                sidx8[lr, pl.ds(v * 16, 16)] = _splat(1 << 30)

            def fbody(vb, soff):
                blk = plsc.load_gather(cand8.at[lr], [_splat(vb)])
                gbase = blk * 128 + lane
                for l in range(8):
                    vals = gblk8[lr, vb, pl.ds(l * 16, 16)]
                    m = vals >= thr
                    at = pl.ds(jnp.minimum(soff, SURV_CAP), 16)
                    plsc.store_compressed(sval8.at[lr, at], vals, mask=m)
                    plsc.store_compressed(sidx8.at[lr, at], gbase + l * 16,
                                          mask=m)
                    cnt = plsc.all_reduce_population_count(m)
                    soff = soff + _scalar(cnt)
                return soff

            n_surv = lax.fori_loop(0, n_blk, fbody, jnp.int32(0))

            def rbody(i, carry2):
                bval = plsc.load_gather(sval8.at[lr], [_splat(i)])
                bidx = plsc.load_gather(sidx8.at[lr], [_splat(i)])
                rank = _splat(0)
                for v in range(SURV_CAP // 16):
                    sv = sval8[lr, pl.ds(v * 16, 16)]
                    si = sidx8[lr, pl.ds(v * 16, 16)]
                    beats = (sv > bval) | ((sv == bval) & (si < bidx))
                    rank = rank + plsc.all_reduce_population_count(beats)
                emit = (lane == 0) & (rank < K_TOP)
                plsc.store_scatter(srow8, [_splat(lr), rank], bval, mask=emit)
                plsc.store_scatter(irow8, [_splat(lr), rank], bidx, mask=emit)
                return carry2

            lax.fori_loop(0, jnp.minimum(n_surv, SURV_CAP), rbody, 0)
            return carry

        lax.fori_loop(0, RB, pass2, 0)

        cps = [pltpu.async_copy(val_hbm.at[irow8.at[lr]], vrows8.at[lr], sem)
               for lr in range(RB)]
        for cp in cps:
            cp.wait()
        pltpu.sync_copy(srow8, sc_o.at[pl.ds(r0, RB)])
        pltpu.sync_copy(irow8, ix_o.at[pl.ds(r0, RB)])
        pltpu.sync_copy(vrows8, gv_o.at[pl.ds(r0, RB)])
        return carry0

    lax.fori_loop(0, rows_per // RB, batch_body, 0)


def _sc_select(s3v, bm, t, values):
    mesh = plsc.VectorSubcoreMesh(core_axis_name="c", subcore_axis_name="s")
    f = pl.kernel(
        _sc_body,
        mesh=mesh,
        compiler_params=pltpu.CompilerParams(needs_layout_passes=False),
        out_type=[
            jax.ShapeDtypeStruct((NQ, K_TOP), jnp.float32),
            jax.ShapeDtypeStruct((NQ, K_TOP), jnp.int32),
            jax.ShapeDtypeStruct((NQ, K_TOP, D), jnp.float32),
        ],
        scratch_types=[
            pltpu.VMEM((NQ // 32,), jnp.float32),
            pltpu.VMEM((RB, NBLK), jnp.float32),
            pltpu.VMEM((RB, CSTRIDE), jnp.int32),
            pltpu.VMEM((16,), jnp.int32),
            pltpu.VMEM((RB, CAND_CAP), jnp.int32),
            pltpu.VMEM((RB, CAND_CAP, D), jnp.float32),
            pltpu.VMEM((RB, SSTRIDE), jnp.float32),
            pltpu.VMEM((RB, SSTRIDE), jnp.int32),
            pltpu.VMEM((RB, K_TOP), jnp.float32),
            pltpu.VMEM((RB, K_TOP), jnp.int32),
            pltpu.VMEM((RB, K_TOP, D), jnp.float32),
            pltpu.SemaphoreType.DMA,
        ],
    )
    return f(s3v, bm, t, values)


def kernel(query, key_memories, value_memories):
    qn = query / (jnp.linalg.norm(query, axis=-1, keepdims=True) + 1e-8)
    kn = key_memories / (
        jnp.linalg.norm(key_memories, axis=-1, keepdims=True) + 1e-8)
    knp = jnp.pad(kn, ((0, N_KEYS_PAD - N_KEYS), (0, 0)))
    s3, bmT3 = _mm(qn, knp)
    bmT = bmT3.reshape(NBLK, NQ)
    t = _thresholds(bmT).reshape(NQ)
    bm = bmT.T
    scores, indices, gathered = _sc_select(
        s3.reshape(NBLK * NQ, D), bm, t, value_memories)
    return (scores, indices, gathered)


# speedup vs baseline: 22.3922x; 1.6949x over previous
import functools

import jax
import jax.numpy as jnp
from jax import lax
from jax.experimental import pallas as pl
from jax.experimental.pallas import tpu as pltpu
from jax.experimental.pallas import tpu_sc as plsc

K_TOP = 32
N_KEYS = 100000
D = 128
NQ = 4096

QBLK = 4096
KBLK = 512
SUB = KBLK // 128
N_KEYS_PAD = ((N_KEYS + KBLK - 1) // KBLK) * KBLK
NBLK = N_KEYS_PAD // 128

CAND_CAP = 48
SURV_CAP = 48
PAD_BLK = NBLK - 1
NEG = -1e30


def _mm_body(q_ref, k_ref, s3_ref, bm_ref):
    ki = pl.program_id(1)
    qn = q_ref[...]
    for b in range(SUB):
        kb = k_ref[b * 128:(b + 1) * 128, :]
        s = lax.dot_general(
            qn, kb, dimension_numbers=(((1,), (1,)), ((), ())),
            preferred_element_type=jnp.float32)
        col = ki * KBLK + b * 128 + lax.broadcasted_iota(jnp.int32, s.shape, 1)
        s = jnp.where(col < N_KEYS, s, NEG)
        s3_ref[b] = s
        st = lax.dot_general(
            kb, qn, dimension_numbers=(((1,), (1,)), ((), ())),
            preferred_element_type=jnp.float32)
        key = ki * KBLK + b * 128 + lax.broadcasted_iota(jnp.int32, st.shape, 0)
        st = jnp.where(key < N_KEYS, st, NEG)
        bm_ref[0, b] = jnp.max(st, axis=0)


def _mm(qn, knp):
    grid = (NQ // QBLK, N_KEYS_PAD // KBLK)
    return pl.pallas_call(
        _mm_body,
        grid=grid,
        in_specs=[
            pl.BlockSpec((QBLK, D), lambda qi, ki: (qi, 0)),
            pl.BlockSpec((KBLK, D), lambda qi, ki: (ki, 0)),
        ],
        out_specs=[
            pl.BlockSpec((SUB, QBLK, D), lambda qi, ki: (ki, qi, 0)),
            pl.BlockSpec((1, SUB, QBLK), lambda qi, ki: (ki, 0, qi)),
        ],
        out_shape=[
            jax.ShapeDtypeStruct((NBLK, NQ, D), jnp.float32),
            jax.ShapeDtypeStruct((NBLK // SUB, SUB, NQ), jnp.float32),
        ],
    )(qn, knp)


TBLK = 256


def _thr_body(bm_ref, t_ref, scr_ref):
    scr_ref[...] = bm_ref[...]

    def body(i, carry):
        b = scr_ref[...]
        m = jnp.max(b, axis=0, keepdims=True)
        scr_ref[...] = jnp.where(b == m, -jnp.inf, b)
        return carry

    lax.fori_loop(0, K_TOP - 1, body, 0)
    t = jnp.max(scr_ref[...], axis=0, keepdims=True)
    t_ref[...] = t - (jnp.abs(t) * 2e-6 + 1e-30)


def _thresholds(bmT):
    return pl.pallas_call(
        _thr_body,
        grid=(NQ // TBLK,),
        in_specs=[pl.BlockSpec((NBLK, TBLK), lambda i: (0, i))],
        out_specs=pl.BlockSpec((1, TBLK), lambda i: (0, i)),
        out_shape=jax.ShapeDtypeStruct((1, NQ), jnp.float32),
        scratch_shapes=[pltpu.VMEM((NBLK, TBLK), jnp.float32)],
    )(bmT)


def _splat(x, dtype=jnp.int32):
    return jnp.full((16,), x, dtype)


def _scalar(vec):
    return lax.reduce_max(vec, axes=(0,))


RB = 8
CSTRIDE = 64
SSTRIDE = 64


def _sc_body(s3_hbm, bm_hbm, t_hbm, val_hbm, sc_o, ix_o, gv_o,
             tch, bmv8, cand8, cnt8, gidx8, gblk8, sval8, sidx8,
             srow8, irow8, vrows8, sem):
    c = lax.axis_index("c")
    s = lax.axis_index("s")
    wid = s * 2 + c
    rows_per = NQ // 32
    base = wid * rows_per
    pltpu.sync_copy(t_hbm.at[pl.ds(base, rows_per)], tch)
    lane = lax.iota(jnp.int32, 16)

    def batch_body(bi, carry0):
        r0 = base + bi * RB
        pltpu.sync_copy(bm_hbm.at[pl.ds(r0, RB)], bmv8)

        def pass1(lr, carry):
            r = r0 + lr
            thr = plsc.load_gather(tch, [_splat(bi * RB + lr)])
            for v in range(CAND_CAP // 16):
                cand8[lr, pl.ds(v * 16, 16)] = _splat(PAD_BLK)

            def cbody(v, off):
                vals = bmv8[lr, pl.ds(v * 16, 16)]
                m = vals >= thr
                ids = lane + v * 16
                plsc.store_compressed(
                    cand8.at[lr, pl.ds(jnp.minimum(off, CAND_CAP), 16)],
                    ids, mask=m)
                cnt = plsc.all_reduce_population_count(m)
                return off + _scalar(cnt)

            cnt_c = lax.fori_loop(0, NBLK // 16, cbody, jnp.int32(0))
            plsc.store_scatter(cnt8, [_splat(lr)],
                               jnp.minimum(_splat(cnt_c), CAND_CAP),
                               mask=lane == 0)
            for v in range(CAND_CAP // 16):
                ids = cand8[lr, pl.ds(v * 16, 16)]
                gidx8[lr, pl.ds(v * 16, 16)] = ids * NQ + r
            return carry

        lax.fori_loop(0, RB, pass1, 0)

        cps = [pltpu.async_copy(s3_hbm.at[gidx8.at[lr]], gblk8.at[lr], sem)
               for lr in range(RB)]
        for cp in cps:
            cp.wait()

        def pass2(lr, carry):
            thr = plsc.load_gather(tch, [_splat(bi * RB + lr)])
            n_blk = _scalar(plsc.load_gather(cnt8, [_splat(lr)]))
            for v in range(SURV_CAP // 16):
                sval8[lr, pl.ds(v * 16, 16)] = jnp.full((16,), NEG,
                                                        jnp.float32)
                sidx8[lr, pl.ds(v * 16, 16)] = _splat(1 << 30)

            def fbody(vb, soff):
                blk = plsc.load_gather(cand8.at[lr], [_splat(vb)])
                gbase = blk * 128 + lane
                for l in range(8):
                    vals = gblk8[lr, vb, pl.ds(l * 16, 16)]
                    m = vals >= thr
                    at = pl.ds(jnp.minimum(soff, SURV_CAP), 16)
                    plsc.store_compressed(sval8.at[lr, at], vals, mask=m)
                    plsc.store_compressed(sidx8.at[lr, at], gbase + l * 16,
                                          mask=m)
                    cnt = plsc.all_reduce_population_count(m)
                    soff = soff + _scalar(cnt)
                return soff

            n_surv = lax.fori_loop(0, n_blk, fbody, jnp.int32(0))

            def rbody(i, carry2):
                bval = plsc.load_gather(sval8.at[lr], [_splat(i)])
                bidx = plsc.load_gather(sidx8.at[lr], [_splat(i)])
                rank = _splat(0)
                for v in range(SURV_CAP // 16):
                    sv = sval8[lr, pl.ds(v * 16, 16)]
                    si = sidx8[lr, pl.ds(v * 16, 16)]
                    beats = (sv > bval) | ((sv == bval) & (si < bidx))
                    rank = rank + plsc.all_reduce_population_count(beats)
                emit = (lane == 0) & (rank < K_TOP)
                plsc.store_scatter(srow8, [_splat(lr), rank], bval, mask=emit)
                plsc.store_scatter(irow8, [_splat(lr), rank], bidx, mask=emit)
                return carry2

            lax.fori_loop(0, jnp.minimum(n_surv, SURV_CAP), rbody, 0)
            return carry

        lax.fori_loop(0, RB, pass2, 0)

        cps = [pltpu.async_copy(val_hbm.at[irow8.at[lr]], vrows8.at[lr], sem)
               for lr in range(RB)]
        for cp in cps:
            cp.wait()
        pltpu.sync_copy(srow8, sc_o.at[pl.ds(r0, RB)])
        pltpu.sync_copy(irow8, ix_o.at[pl.ds(r0, RB)])
        pltpu.sync_copy(vrows8, gv_o.at[pl.ds(r0, RB)])
        return carry0

    lax.fori_loop(0, rows_per // RB, batch_body, 0)


def _sc_select(s3v, bm, t, values):
    mesh = plsc.VectorSubcoreMesh(core_axis_name="c", subcore_axis_name="s")
    f = pl.kernel(
        _sc_body,
        mesh=mesh,
        compiler_params=pltpu.CompilerParams(needs_layout_passes=False),
        out_type=[
            jax.ShapeDtypeStruct((NQ, K_TOP), jnp.float32),
            jax.ShapeDtypeStruct((NQ, K_TOP), jnp.int32),
            jax.ShapeDtypeStruct((NQ, K_TOP, D), jnp.float32),
        ],
        scratch_types=[
            pltpu.VMEM((NQ // 32,), jnp.float32),
            pltpu.VMEM((RB, NBLK), jnp.float32),
            pltpu.VMEM((RB, CSTRIDE), jnp.int32),
            pltpu.VMEM((16,), jnp.int32),
            pltpu.VMEM((RB, CAND_CAP), jnp.int32),
            pltpu.VMEM((RB, CAND_CAP, D), jnp.float32),
            pltpu.VMEM((RB, SSTRIDE), jnp.float32),
            pltpu.VMEM((RB, SSTRIDE), jnp.int32),
            pltpu.VMEM((RB, K_TOP), jnp.float32),
            pltpu.VMEM((RB, K_TOP), jnp.int32),
            pltpu.VMEM((RB, K_TOP, D), jnp.float32),
            pltpu.SemaphoreType.DMA,
        ],
    )
    return f(s3v, bm, t, values)


def kernel(query, key_memories, value_memories):
    qn = query / (jnp.linalg.norm(query, axis=-1, keepdims=True) + 1e-8)
    kn = key_memories / (
        jnp.linalg.norm(key_memories, axis=-1, keepdims=True) + 1e-8)
    knp = jnp.pad(kn, ((0, N_KEYS_PAD - N_KEYS), (0, 0)))
    s3, bmT3 = _mm(qn, knp)
    bmT = bmT3.reshape(NBLK, NQ)
    t = _thresholds(bmT).reshape(NQ)
    bm = bmT.T
    scores, indices, gathered = _sc_select(
        s3.reshape(NBLK * NQ, D), bm, t, value_memories)
    return (scores, indices, gathered)
